# baseline (speedup 1.0000x reference)
import functools

import jax
import jax.numpy as jnp
from jax import lax
from jax.experimental import pallas as pl
from jax.experimental.pallas import tpu as pltpu
from jax.experimental.pallas import tpu_sc as plsc

B = 16384
BARS = 8
BINS = 8
ADIM = 32
NEMB = 512
ROWS = B * BARS
NTT = BINS ** 4

NC = 2
NS = 16
NW = NC * NS
RPW = ROWS // NW
CHUNK = 32
NBUF = 4
NCHUNK = RPW // CHUNK


def _tables_body(e0, e1, e2, e3, w, b, tt_ref):
    r = lax.broadcasted_iota(jnp.int32, (64, 8), 0)
    c = lax.broadcasted_iota(jnp.int32, (64, 8), 1)
    s_lo = (c == (r & 7)).astype(jnp.float32)
    s_hi = (c == (r >> 3)).astype(jnp.float32)
    wv = w[...]
    dot = functools.partial(jnp.dot, preferred_element_type=jnp.float32)
    t0 = dot(e0[...], wv[0:32])
    t1 = dot(e1[...], wv[32:64])
    t2 = dot(e2[...], wv[64:96])
    t3 = dot(e3[...], wv[96:128])
    t01 = dot(s_lo, t0) + dot(s_hi, t1)
    t23 = dot(s_lo, t2) + dot(s_hi, t3) + b[...]
    pr = lax.broadcasted_iota(jnp.int32, (NEMB, NEMB // 2), 0)
    pc = lax.broadcasted_iota(jnp.int32, (NEMB, NEMB // 2), 1)
    src = 32 * (pc >> 4) + (pc & 15)
    p_lo = (pr == src).astype(jnp.float32)
    p_hi = (pr == src + 16).astype(jnp.float32)
    rr = lax.broadcasted_iota(jnp.int32, (NTT, 64), 0)
    cc = lax.broadcasted_iota(jnp.int32, (NTT, 64), 1)
    g_lo = (cc == (rr & 63)).astype(jnp.float32)
    g_hi = (cc == (rr >> 6)).astype(jnp.float32)
    lo = dot(g_lo, dot(t01, p_lo)) + dot(g_hi, dot(t23, p_lo))
    hi = dot(g_lo, dot(t01, p_hi)) + dot(g_hi, dot(t23, p_hi))

    def bf16_bits(x):
        bu = lax.bitcast_convert_type(x, jnp.uint32)
        return (bu + 0x7FFF + ((bu >> 16) & 1)) >> 16

    packed = bf16_bits(lo) | (bf16_bits(hi) << 16)
    tt_ref[...] = lax.bitcast_convert_type(packed, jnp.int32)


def _build_table(E0, E1, E2, E3, W, b2):
    return pl.pallas_call(
        _tables_body,
        out_shape=jax.ShapeDtypeStruct((NTT, NEMB // 2), jnp.int32),
    )(E0, E1, E2, E3, W, b2)


def _idx_body(a_ref, idx_ref):
    idx_ref[...] = (a_ref[:, 0, :] + 8 * a_ref[:, 1, :]
                    + 64 * a_ref[:, 2, :] + 512 * a_ref[:, 3, :])


def _build_idx(a_t):
    return pl.pallas_call(
        _idx_body,
        out_shape=jax.ShapeDtypeStruct((BARS, B), jnp.int32),
    )(a_t)


BPW = B // (NW // BARS)


def _gather_body(tt_hbm, idxt_hbm, out_hbm, idx_v, *rest):
    bbufs = rest[:NBUF]
    fbufs = rest[NBUF:2 * NBUF]
    gsems = rest[2 * NBUF:3 * NBUF]
    osems = rest[3 * NBUF:4 * NBUF]
    cid = lax.axis_index("c")
    sid = lax.axis_index("s")
    wid = sid * NC + cid
    bar = wid // (NW // BARS)
    bslot = wid % (NW // BARS)
    b0w = bslot * BPW

    pltpu.sync_copy(idxt_hbm.at[bar, pl.ds(b0w, BPW)], idx_v)

    def gather(g, buf, gs):
        return pltpu.make_async_copy(
            tt_hbm.at[idx_v.at[pl.ds(g * CHUNK, CHUNK)]], buf, gs)

    def writeout(g, buf, os):
        return pltpu.make_async_copy(
            buf, out_hbm.at[pl.ds(b0w + g * CHUNK, CHUNK), bar], os)

    def widen(bbuf, fbuf):
        def colblk(t, carry):
            for r in range(CHUNK):
                w = bbuf[r, pl.ds(16 * t, 16)]
                fbuf[r, pl.ds(32 * t, 16)] = w << 16
                fbuf[r, pl.ds(32 * t + 16, 16)] = w & jnp.int32(-65536)
            return carry
        lax.fori_loop(0, NEMB // 32, colblk, 0)

    for j in range(NBUF):
        gather(j, bbufs[j], gsems[j]).start()

    def slot(g, bbuf, fbuf, gs, os):
        gather(g, bbuf, gs).wait()

        @pl.when(g >= NBUF)
        def _():
            writeout(g - NBUF, fbuf, os).wait()
        widen(bbuf, fbuf)
        writeout(g, fbuf, os).start()

        @pl.when(g + NBUF < NCHUNK)
        def _():
            gather(g + NBUF, bbuf, gs).start()

    def body(i, carry):
        for j in range(NBUF):
            slot(NBUF * i + j, bbufs[j], fbufs[j], gsems[j], osems[j])
        return carry

    lax.fori_loop(0, NCHUNK // NBUF, body, 0)

    for j in range(NBUF):
        writeout(NCHUNK - NBUF + j, fbufs[j], osems[j]).wait()


@functools.partial(
    pl.kernel,
    out_type=jax.ShapeDtypeStruct((B, BARS, NEMB), jnp.int32),
    mesh=plsc.VectorSubcoreMesh(core_axis_name="c", subcore_axis_name="s"),
    scratch_types=(
        [pltpu.VMEM((BPW,), jnp.int32)]
        + [pltpu.VMEM((CHUNK, NEMB // 2), jnp.int32)] * NBUF
        + [pltpu.VMEM((CHUNK, NEMB), jnp.int32)] * NBUF
        + [pltpu.SemaphoreType.DMA] * (2 * NBUF)
    ),
)
def _gather_rows(tt_hbm, idxt_hbm, out_hbm, *rest):
    _gather_body(tt_hbm, idxt_hbm, out_hbm, *rest)


@jax.jit
def kernel(attributes, E0, E1, E2, E3, W, b):
    a_t = attributes.astype(jnp.int32).transpose(1, 2, 0)
    tt = _build_table(E0, E1, E2, E3, W, b.reshape(1, NEMB))
    idxt = _build_idx(a_t)
    out_bits = _gather_rows(tt, idxt)
    return lax.bitcast_convert_type(out_bits, jnp.float32)


# speedup vs baseline: 1.4872x; 1.4872x over previous
import functools

import jax
import jax.numpy as jnp
from jax import lax
from jax.experimental import pallas as pl
from jax.experimental.pallas import tpu as pltpu
from jax.experimental.pallas import tpu_sc as plsc

B = 16384
BARS = 8
BINS = 8
ADIM = 32
NEMB = 512
ROWS = B * BARS
NTT = BINS ** 4

NC = 2
NS = 16
NW = NC * NS
RPW = ROWS // NW
CHUNK = 32
NBUF = 4
NCHUNK = RPW // CHUNK


def _tables_body(e0, e1, e2, e3, w, b, tt_ref):
    r = lax.broadcasted_iota(jnp.int32, (64, 8), 0)
    c = lax.broadcasted_iota(jnp.int32, (64, 8), 1)
    s_lo = (c == (r & 7)).astype(jnp.float32)
    s_hi = (c == (r >> 3)).astype(jnp.float32)
    wv = w[...]
    dot = functools.partial(jnp.dot, preferred_element_type=jnp.float32)
    t0 = dot(e0[...], wv[0:32])
    t1 = dot(e1[...], wv[32:64])
    t2 = dot(e2[...], wv[64:96])
    t3 = dot(e3[...], wv[96:128])
    t01 = dot(s_lo, t0) + dot(s_hi, t1)
    t23 = dot(s_lo, t2) + dot(s_hi, t3) + b[...]
    pr = lax.broadcasted_iota(jnp.int32, (NEMB, NEMB // 2), 0)
    pc = lax.broadcasted_iota(jnp.int32, (NEMB, NEMB // 2), 1)
    src = 32 * (pc >> 4) + (pc & 15)
    p_lo = (pr == src).astype(jnp.float32)
    p_hi = (pr == src + 16).astype(jnp.float32)
    rr = lax.broadcasted_iota(jnp.int32, (NTT, 64), 0)
    cc = lax.broadcasted_iota(jnp.int32, (NTT, 64), 1)
    g_lo = (cc == (rr & 63)).astype(jnp.float32)
    g_hi = (cc == (rr >> 6)).astype(jnp.float32)
    lo = dot(g_lo, dot(t01, p_lo)) + dot(g_hi, dot(t23, p_lo))
    hi = dot(g_lo, dot(t01, p_hi)) + dot(g_hi, dot(t23, p_hi))

    def bf16_bits(x):
        bu = lax.bitcast_convert_type(x, jnp.uint32)
        return (bu + 0x7FFF + ((bu >> 16) & 1)) >> 16

    packed = bf16_bits(lo) | (bf16_bits(hi) << 16)
    tt_ref[...] = lax.bitcast_convert_type(packed, jnp.int32)


def _build_table(E0, E1, E2, E3, W, b2):
    return pl.pallas_call(
        _tables_body,
        out_shape=jax.ShapeDtypeStruct((NTT, NEMB // 2), jnp.int32),
    )(E0, E1, E2, E3, W, b2)


def _idx_body(a_ref, idx_ref):
    idx_ref[...] = (a_ref[:, 0, :] + 8 * a_ref[:, 1, :]
                    + 64 * a_ref[:, 2, :] + 512 * a_ref[:, 3, :])


def _build_idx(a_t):
    return pl.pallas_call(
        _idx_body,
        out_shape=jax.ShapeDtypeStruct((BARS, B), jnp.int32),
    )(a_t)


BPW = B // (NW // BARS)


def _gather_body(tt_hbm, idxt_hbm, out_hbm, idx_v, *rest):
    bbufs = rest[:NBUF]
    fbufs = rest[NBUF:2 * NBUF]
    gsems = rest[2 * NBUF:3 * NBUF]
    osems = rest[3 * NBUF:4 * NBUF]
    cid = lax.axis_index("c")
    sid = lax.axis_index("s")
    wid = sid * NC + cid
    bar = wid // (NW // BARS)
    bslot = wid % (NW // BARS)
    b0w = bslot * BPW

    pltpu.sync_copy(idxt_hbm.at[bar, pl.ds(b0w, BPW)], idx_v)

    def gather(g, buf, gs):
        return pltpu.make_async_copy(
            tt_hbm.at[idx_v.at[pl.ds(g * CHUNK, CHUNK)]], buf, gs)

    def writeout(g, buf, os):
        return pltpu.make_async_copy(
            buf, out_hbm.at[pl.ds(b0w + g * CHUNK, CHUNK), bar], os)

    def widen(bbuf, fbuf):
        @functools.partial(plsc.parallel_loop, 0, NEMB // 32, unroll=4)
        def colblk(t):
            for r in range(CHUNK):
                w = bbuf[r, pl.ds(16 * t, 16)]
                fbuf[r, pl.ds(32 * t, 16)] = w << 16
                fbuf[r, pl.ds(32 * t + 16, 16)] = w & jnp.int32(-65536)

    for j in range(NBUF):
        gather(j, bbufs[j], gsems[j]).start()

    def slot(g, bbuf, fbuf, gs, os):
        gather(g, bbuf, gs).wait()

        @pl.when(g >= NBUF)
        def _():
            writeout(g - NBUF, fbuf, os).wait()
        widen(bbuf, fbuf)
        writeout(g, fbuf, os).start()

        @pl.when(g + NBUF < NCHUNK)
        def _():
            gather(g + NBUF, bbuf, gs).start()

    def body(i, carry):
        for j in range(NBUF):
            slot(NBUF * i + j, bbufs[j], fbufs[j], gsems[j], osems[j])
        return carry

    lax.fori_loop(0, NCHUNK // NBUF, body, 0)

    for j in range(NBUF):
        writeout(NCHUNK - NBUF + j, fbufs[j], osems[j]).wait()


@functools.partial(
    pl.kernel,
    out_type=jax.ShapeDtypeStruct((B, BARS, NEMB), jnp.int32),
    mesh=plsc.VectorSubcoreMesh(core_axis_name="c", subcore_axis_name="s"),
    scratch_types=(
        [pltpu.VMEM((BPW,), jnp.int32)]
        + [pltpu.VMEM((CHUNK, NEMB // 2), jnp.int32)] * NBUF
        + [pltpu.VMEM((CHUNK, NEMB), jnp.int32)] * NBUF
        + [pltpu.SemaphoreType.DMA] * (2 * NBUF)
    ),
)
def _gather_rows(tt_hbm, idxt_hbm, out_hbm, *rest):
    _gather_body(tt_hbm, idxt_hbm, out_hbm, *rest)


@jax.jit
def kernel(attributes, E0, E1, E2, E3, W, b):
    a_t = attributes.astype(jnp.int32).transpose(1, 2, 0)
    tt = _build_table(E0, E1, E2, E3, W, b.reshape(1, NEMB))
    idxt = _build_idx(a_t)
    out_bits = _gather_rows(tt, idxt)
    return lax.bitcast_convert_type(out_bits, jnp.float32)


# speedup vs baseline: 2.4195x; 1.6269x over previous
import functools

import jax
import jax.numpy as jnp
from jax import lax
from jax.experimental import pallas as pl
from jax.experimental.pallas import tpu as pltpu
from jax.experimental.pallas import tpu_sc as plsc

B = 16384
BARS = 8
BINS = 8
ADIM = 32
NEMB = 512
ROWS = B * BARS
NTT = BINS ** 4

NC = 2
NS = 16
NW = NC * NS
RPW = ROWS // NW
CHUNK = 32
NBUF = 4
NCHUNK = RPW // CHUNK


def _tables_body(e0, e1, e2, e3, w, b, tt_ref):
    r = lax.broadcasted_iota(jnp.int32, (64, 8), 0)
    c = lax.broadcasted_iota(jnp.int32, (64, 8), 1)
    s_lo = (c == (r & 7)).astype(jnp.float32)
    s_hi = (c == (r >> 3)).astype(jnp.float32)
    wv = w[...]
    dot = functools.partial(jnp.dot, preferred_element_type=jnp.float32)
    t0 = dot(e0[...], wv[0:32])
    t1 = dot(e1[...], wv[32:64])
    t2 = dot(e2[...], wv[64:96])
    t3 = dot(e3[...], wv[96:128])
    t01 = dot(s_lo, t0) + dot(s_hi, t1)
    t23 = dot(s_lo, t2) + dot(s_hi, t3) + b[...]
    rr = lax.broadcasted_iota(jnp.int32, (NTT, 64), 0)
    cc = lax.broadcasted_iota(jnp.int32, (NTT, 64), 1)
    g_lo = (cc == (rr & 63)).astype(jnp.float32)
    g_hi = (cc == (rr >> 6)).astype(jnp.float32)
    tt_ref[...] = dot(g_lo, t01) + dot(g_hi, t23)


def _build_table(E0, E1, E2, E3, W, b2):
    return pl.pallas_call(
        _tables_body,
        out_shape=jax.ShapeDtypeStruct((NTT, NEMB), jnp.float32),
    )(E0, E1, E2, E3, W, b2)


def _idx_body(a_ref, idx_ref):
    idx_ref[...] = (a_ref[:, 0, :] + 8 * a_ref[:, 1, :]
                    + 64 * a_ref[:, 2, :] + 512 * a_ref[:, 3, :])


def _build_idx(a_t):
    return pl.pallas_call(
        _idx_body,
        out_shape=jax.ShapeDtypeStruct((BARS, B), jnp.int32),
    )(a_t)


BPW = B // (NW // BARS)


def _gather_body(tt_hbm, idxt_hbm, out_hbm, idx_v, *rest):
    bufs = rest[:NBUF]
    gsems = rest[NBUF:2 * NBUF]
    osems = rest[2 * NBUF:3 * NBUF]
    cid = lax.axis_index("c")
    sid = lax.axis_index("s")
    wid = sid * NC + cid
    bar = wid // (NW // BARS)
    bslot = wid % (NW // BARS)
    b0w = bslot * BPW

    pltpu.sync_copy(idxt_hbm.at[bar, pl.ds(b0w, BPW)], idx_v)

    def gather(g, buf, gs):
        return pltpu.make_async_copy(
            tt_hbm.at[idx_v.at[pl.ds(g * CHUNK, CHUNK)]], buf, gs)

    def writeout(g, buf, os):
        return pltpu.make_async_copy(
            buf, out_hbm.at[pl.ds(b0w + g * CHUNK, CHUNK), bar], os)

    for j in range(NBUF):
        gather(j, bufs[j], gsems[j]).start()

    def slot(g, buf, gs, os):
        gather(g, buf, gs).wait()
        writeout(g, buf, os).start()

        @pl.when(g + NBUF < NCHUNK)
        def _():
            writeout(g, buf, os).wait()
            gather(g + NBUF, buf, gs).start()

    def body(i, carry):
        for j in range(NBUF):
            slot(NBUF * i + j, bufs[j], gsems[j], osems[j])
        return carry

    lax.fori_loop(0, NCHUNK // NBUF, body, 0)

    for j in range(NBUF):
        writeout(NCHUNK - NBUF + j, bufs[j], osems[j]).wait()


@functools.partial(
    pl.kernel,
    out_type=jax.ShapeDtypeStruct((B, BARS, NEMB), jnp.float32),
    mesh=plsc.VectorSubcoreMesh(core_axis_name="c", subcore_axis_name="s"),
    scratch_types=(
        [pltpu.VMEM((BPW,), jnp.int32)]
        + [pltpu.VMEM((CHUNK, NEMB), jnp.float32)] * NBUF
        + [pltpu.SemaphoreType.DMA] * (2 * NBUF)
    ),
)
def _gather_rows(tt_hbm, idxt_hbm, out_hbm, *rest):
    _gather_body(tt_hbm, idxt_hbm, out_hbm, *rest)


@jax.jit
def kernel(attributes, E0, E1, E2, E3, W, b):
    a_t = attributes.astype(jnp.int32).transpose(1, 2, 0)
    tt = _build_table(E0, E1, E2, E3, W, b.reshape(1, NEMB))
    idxt = _build_idx(a_t)
    return _gather_rows(tt, idxt)
